# Initial kernel scaffold; baseline (speedup 1.0000x reference)
import jax
import jax.numpy as jnp
from jax.experimental import pallas as pl


def kernel(img1, img2):
    raise NotImplementedError("write your pallas kernel here")


# speedup vs baseline: 41.4086x; 41.4086x over previous
import functools

import jax
import jax.numpy as jnp
from jax import lax
from jax.experimental import pallas as pl
from jax.experimental.pallas import tpu as pltpu
from jax.experimental.pallas import tpu_sc as plsc

NBINS = 256
LANES = 16
NC = 2
NS = 16
NW = NC * NS
ELEMS = 32 * 3 * 512 * 512
PER_W = ELEMS // NW
CHUNK = 32768
NCHUNKS = PER_W // CHUNK
SUBH = LANES * NBINS
UNROLL = 8

_MESH = plsc.VectorSubcoreMesh(core_axis_name="c", subcore_axis_name="s")


@functools.partial(
    pl.kernel,
    out_type=jax.ShapeDtypeStruct((6, NW, NBINS), jnp.float32),
    mesh=_MESH,
    compiler_params=pltpu.CompilerParams(needs_layout_passes=False),
    scratch_types=[
        pltpu.VMEM((CHUNK,), jnp.float32),
        pltpu.VMEM((CHUNK,), jnp.float32),
        pltpu.VMEM((6 * SUBH,), jnp.float32),
        pltpu.VMEM((6 * NBINS,), jnp.float32),
        pltpu.SemaphoreType.DMA,
        pltpu.SemaphoreType.DMA,
    ],
)
def _sc_hist(i1, i2, out, buf0, buf1, hist, hout, sem0, sem1):
    w = lax.axis_index("s") * NC + lax.axis_index("c")
    base = w * PER_W
    lane_off = lax.broadcasted_iota(jnp.int32, (LANES,), 0) * NBINS
    ones = jnp.ones((LANES,), jnp.float32)
    zeros = jnp.zeros((LANES,), jnp.float32)

    def _zero(i, _):
        hist[pl.ds(i * LANES, LANES)] = zeros
        return 0

    lax.fori_loop(0, 6 * SUBH // LANES, _zero, 0)

    bufs = (buf0, buf1)
    sems = (sem0, sem1)

    for img, ref in ((0, i1), (1, i2)):
        pltpu.async_copy(ref.at[pl.ds(base, CHUNK)], buf0, sem0)
        pltpu.async_copy(ref.at[pl.ds(base + CHUNK, CHUNK)], buf1, sem1)

        def _pair(p, _, ref=ref, img=img):
            for b in range(2):
                t = p * 2 + b
                cur, sem = bufs[b], sems[b]
                pltpu.make_async_copy(
                    ref.at[pl.ds(base, CHUNK)], cur, sem).wait()
                off = lane_off + (img * 3 + t // 8) * SUBH

                def _vec(i, _, cur=cur, off=off):
                    vbase = i * (LANES * UNROLL)
                    for u in range(UNROLL):
                        x = cur[pl.ds(vbase + u * LANES, LANES)]
                        f = jnp.minimum(x * 256.0, 255.0)
                        idx = f.astype(jnp.int32) + off
                        plsc.addupdate_scatter(hist, [idx], ones)
                    return 0

                lax.fori_loop(0, CHUNK // (LANES * UNROLL), _vec, 0)

                @pl.when(t + 2 < NCHUNKS)
                def _(ref=ref, cur=cur, sem=sem, t=t):
                    pltpu.async_copy(
                        ref.at[pl.ds(base + (t + 2) * CHUNK, CHUNK)],
                        cur, sem)

            return 0

        lax.fori_loop(0, NCHUNKS // 2, _pair, 0)

    def _drain(r, _):
        reg = r // 16
        g = r % 16
        acc = zeros
        for s in range(LANES):
            acc = acc + hist[pl.ds(reg * SUBH + s * NBINS + g * LANES, LANES)]
        hout[pl.ds(reg * NBINS + g * LANES, LANES)] = acc
        return 0

    lax.fori_loop(0, 6 * 16, _drain, 0)

    for r in range(6):
        pltpu.sync_copy(hout.at[pl.ds(r * NBINS, NBINS)], out.at[r, w])


def _tc_loss(h_ref, o_ref):
    h = h_ref[...]
    hs = jnp.sum(h, axis=1)
    tot = jnp.sum(hs, axis=1, keepdims=True)
    hn = hs / tot
    r = lax.broadcasted_iota(jnp.int32, (NBINS, NBINS), 0)
    c = lax.broadcasted_iota(jnp.int32, (NBINS, NBINS), 1)
    m = (r <= c).astype(jnp.float32)
    cdf = lax.dot(hn, m, precision=lax.Precision.HIGHEST,
                  preferred_element_type=jnp.float32)
    d = cdf[0:3, :] - cdf[3:6, :]
    loss = jnp.sum(jnp.abs(d)) / 3.0
    o_ref[...] = jnp.broadcast_to(loss, (1, 1))


def kernel(img1, img2):
    part = _sc_hist(img1.reshape(-1), img2.reshape(-1))
    out = pl.pallas_call(
        _tc_loss,
        out_shape=jax.ShapeDtypeStruct((1, 1), jnp.float32),
    )(part)
    return out[0, 0]
